# baseline (speedup 1.0000x reference)
import jax
import jax.numpy as jnp
from jax import lax
from jax.experimental import pallas as pl
from jax.experimental.pallas import tpu as pltpu
from jax.experimental.pallas import tpu_sc as plsc

D = 64
S = 200
NW = 32
CHUNK = 256
GW = 128
PPW = 6400
NCH = PPW // CHUNK


def _body(xf_hbm, tbl_hbm, pe_hbm, out_hbm,
          idx_v, idx1, par1, pe_v, buf0, buf1, obuf, gsem0, gsem1):
    wid = lax.axis_index("s") * 2 + lax.axis_index("c")
    base = wid * PPW
    pbase = base // 2

    pltpu.sync_copy(xf_hbm.at[wid], idx_v)
    pltpu.sync_copy(pe_hbm, pe_v)

    def flat_idx(i, carry):
        r = idx_v[i // 8, pl.ds((i % 8) * 16, 16)]
        idx1[pl.ds(i * 16, 16)] = lax.shift_right_logical(r, 1)
        par1[pl.ds(i * 16, 16)] = r & 1
        return carry
    lax.fori_loop(0, (PPW // GW) * 8, flat_idx, 0)

    def gather_chunk(c, buf, sem):
        for k in range(CHUNK // GW):
            pltpu.async_copy(
                tbl_hbm.at[idx1.at[pl.ds(pl.multiple_of(c * CHUNK + k * GW, 8), GW)]],
                buf.at[pl.ds(k * GW, GW)],
                sem,
            )

    def drain_chunk(buf, sem):
        for k in range(CHUNK // GW):
            pltpu.make_async_copy(
                tbl_hbm.at[idx1.at[pl.ds(k * GW, GW)]],
                buf.at[pl.ds(k * GW, GW)],
                sem,
            ).wait()

    def process_chunk(c, buf):
        def ubody(u, sp):
            for h in range(2):
                pos = 2 * u + h
                psplat = plsc.load_gather(
                    par1, [jax.lax.broadcast(c * CHUNK + pos, (16,))]
                )
                m = psplat == 1
                for k in range(D // 16):
                    lo = buf[pos, pl.ds(k * 16, 16)]
                    hi = buf[pos, pl.ds(D + k * 16, 16)]
                    pej = pe_v[sp, pl.ds(h * D + k * 16, 16)]
                    obuf[u, pl.ds(h * D + k * 16, 16)] = (
                        jnp.where(m, hi, lo) + pej
                    )
            return lax.select(sp + 1 == S // 2, 0, sp + 1)

        sp0 = ((base + c * CHUNK) // 2) % (S // 2)
        lax.fori_loop(0, CHUNK // 2, ubody, sp0)
        pltpu.sync_copy(
            obuf,
            out_hbm.at[pl.ds(pl.multiple_of(pbase + c * (CHUNK // 2), 8), CHUNK // 2)],
        )

    gather_chunk(0, buf0, gsem0)

    def pair_body(t, carry):
        c0 = 2 * t
        gather_chunk(c0 + 1, buf1, gsem1)
        drain_chunk(buf0, gsem0)
        process_chunk(c0, buf0)

        @pl.when(c0 + 2 < NCH)
        def _():
            gather_chunk(c0 + 2, buf0, gsem0)

        drain_chunk(buf1, gsem1)
        process_chunk(c0 + 1, buf1)
        return carry

    lax.fori_loop(0, NCH // 2, pair_body, 0)

    drain_chunk(buf0, gsem0)
    process_chunk(NCH - 1, buf0)


def kernel(x, table, pe):
    b, s = x.shape
    rows = b * s
    xf = x.reshape(NW, PPW // GW, GW)
    tbl2 = table.reshape(table.shape[0] // 2, 2 * D)
    pe2 = pe[:s].reshape(s // 2, 2 * D)

    mesh = plsc.VectorSubcoreMesh(core_axis_name="c", subcore_axis_name="s")
    out2 = pl.kernel(
        _body,
        out_type=jax.ShapeDtypeStruct((rows // 2, 2 * D), jnp.float32),
        mesh=mesh,
        compiler_params=pltpu.CompilerParams(
            use_tc_tiling_on_sc=True, needs_layout_passes=False
        ),
        scratch_types=[
            pltpu.VMEM((PPW // GW, GW), jnp.int32),
            pltpu.VMEM((PPW,), jnp.int32),
            pltpu.VMEM((PPW,), jnp.int32),
            pltpu.VMEM((S // 2, 2 * D), jnp.float32),
            pltpu.VMEM((CHUNK, 2 * D), jnp.float32),
            pltpu.VMEM((CHUNK, 2 * D), jnp.float32),
            pltpu.VMEM((CHUNK // 2, 2 * D), jnp.float32),
            pltpu.SemaphoreType.DMA,
            pltpu.SemaphoreType.DMA,
        ],
    )(xf, tbl2, pe2)
    return out2.reshape(b, s, D)


# speedup vs baseline: 1.1597x; 1.1597x over previous
import jax
import jax.numpy as jnp
from jax import lax
from jax.experimental import pallas as pl
from jax.experimental.pallas import tpu as pltpu
from jax.experimental.pallas import tpu_sc as plsc

D = 64
S = 200
NW = 32
SUB = 100
SEQ_PER_CHUNK = 2
CHUNK = SEQ_PER_CHUNK * S
SUBS_PER_CHUNK = CHUNK // SUB


def _body(xf_hbm, table_hbm, pe_hbm, out_hbm,
          idx_v, pe_v, rows0, rows1, gsem0, gsem1):
    nsub = xf_hbm.shape[1]
    rows_per_worker = nsub * SUB
    nchunk = rows_per_worker // CHUNK

    wid = lax.axis_index("s") * 2 + lax.axis_index("c")
    base = wid * rows_per_worker
    bseq = wid * (rows_per_worker // S)

    pltpu.sync_copy(xf_hbm.at[wid], idx_v)
    pltpu.sync_copy(pe_hbm, pe_v)

    def gather_chunk(c, buf, sem):
        for k in range(SUBS_PER_CHUNK):
            pltpu.async_copy(
                table_hbm.at[idx_v.at[c * SUBS_PER_CHUNK + k]],
                buf.at[k // 2, pl.ds((k % 2) * SUB, SUB)],
                sem,
            )

    def drain_chunk(buf, sem):
        for k in range(SUBS_PER_CHUNK):
            pltpu.make_async_copy(
                table_hbm.at[idx_v.at[k]],
                buf.at[k // 2, pl.ds((k % 2) * SUB, SUB)],
                sem,
            ).wait()

    def add_pe(buf):
        def jbody(j, carry):
            for c2 in range(SEQ_PER_CHUNK):
                for k in range(D // 16):
                    pv = pe_v[j, pl.ds(k * 16, 16)]
                    plsc.addupdate(buf.at[c2, j, pl.ds(k * 16, 16)], pv)
            return carry
        lax.fori_loop(0, S, jbody, 0)

    gather_chunk(0, rows0, gsem0)

    def pair_body(t, carry):
        c0 = 2 * t
        gather_chunk(c0 + 1, rows1, gsem1)
        drain_chunk(rows0, gsem0)
        add_pe(rows0)
        pltpu.sync_copy(rows0, out_hbm.at[pl.ds(bseq + c0 * SEQ_PER_CHUNK, SEQ_PER_CHUNK)])

        @pl.when(t < nchunk // 2 - 1)
        def _():
            gather_chunk(c0 + 2, rows0, gsem0)

        drain_chunk(rows1, gsem1)
        add_pe(rows1)
        pltpu.sync_copy(rows1, out_hbm.at[pl.ds(bseq + (c0 + 1) * SEQ_PER_CHUNK, SEQ_PER_CHUNK)])
        return carry

    lax.fori_loop(0, nchunk // 2, pair_body, 0)


def kernel(x, table, pe):
    b, s = x.shape
    rows = b * s
    nsub = rows // (NW * SUB)
    xf = x.reshape(NW, nsub, SUB)
    pe_s = pe[:s]

    mesh = plsc.VectorSubcoreMesh(core_axis_name="c", subcore_axis_name="s")
    out = pl.kernel(
        _body,
        out_type=jax.ShapeDtypeStruct((b, s, D), jnp.float32),
        mesh=mesh,
        compiler_params=pltpu.CompilerParams(use_tc_tiling_on_sc=False),
        scratch_types=[
            pltpu.VMEM((nsub, SUB), jnp.int32),
            pltpu.VMEM((S, D), jnp.float32),
            pltpu.VMEM((SEQ_PER_CHUNK, S, D), jnp.float32),
            pltpu.VMEM((SEQ_PER_CHUNK, S, D), jnp.float32),
            pltpu.SemaphoreType.DMA,
            pltpu.SemaphoreType.DMA,
        ],
    )(xf, table, pe_s)
    return out
